# Initial kernel scaffold; baseline (speedup 1.0000x reference)
import jax
import jax.numpy as jnp
from jax.experimental import pallas as pl


def kernel(tx_feats, emb_card, emb_merch, tc_src, tc_dst, tm_src, tm_dst, params):
    raise NotImplementedError("write your pallas kernel here")


# speedup vs baseline: 2.2460x; 2.2460x over previous
import functools

import jax
import jax.numpy as jnp
from jax import lax
from jax.experimental import pallas as pl
from jax.experimental.pallas import tpu as pltpu
from jax.experimental.pallas import tpu_sc as plsc

_D = 128
_NCHUNK = 4
_CW = 32
_B = 512
_K = 4
_NB = 608
_EP = _NB * _B
_R = 1000


def _pad128(n):
    return ((n + 1 + 127) // 128) * 128


def _segsum_call(table, s3, d3, zrows, n_src, n_dst):
    n_dst_pad = _pad128(n_dst)
    nz = n_dst_pad // 16
    nbt = _NB // 16
    mesh = plsc.VectorSubcoreMesh(core_axis_name="c", subcore_axis_name="s")

    @functools.partial(
        pl.kernel,
        out_type=jax.ShapeDtypeStruct((_NCHUNK, n_dst_pad, _CW), jnp.float32),
        mesh=mesh,
        scratch_types=[
            pltpu.VMEM((_K, 128), jnp.int32),
            pltpu.VMEM((_K, 128), jnp.int32),
            pltpu.VMEM((_K, 128, _CW), jnp.float32),
            pltpu.VMEM_SHARED((n_dst_pad, _CW), jnp.float32),
            pltpu.SemaphoreType.DMA,
        ],
        compiler_params=pltpu.CompilerParams(use_tc_tiling_on_sc=False),
    )
    def k(tbl, s_idx, d_idx, zr, out, sbuf, dbuf, rbuf, acc, sem):
        cid = lax.axis_index("c")
        sid = lax.axis_index("s")
        for p in range(2):
            chunk = cid * 2 + p
            pltpu.sync_copy(zr, acc.at[pl.ds(sid * nz, nz)])
            plsc.subcore_barrier()

            def body(i, carry):
                b = sid * nbt + i
                pltpu.sync_copy(s_idx.at[b], sbuf)
                pltpu.sync_copy(d_idx.at[b], dbuf)
                cps = [
                    pltpu.async_copy(tbl.at[chunk].at[sbuf.at[kk]],
                                     rbuf.at[kk], sem)
                    for kk in range(_K)
                ]
                for cp in cps:
                    cp.wait()
                for kk in range(_K):
                    pltpu.sync_copy(rbuf.at[kk], acc.at[dbuf.at[kk]], add=True)
                return carry

            lax.fori_loop(0, nbt, body, 0)
            plsc.subcore_barrier()
            pltpu.sync_copy(acc.at[pl.ds(sid * nz, nz)],
                            out.at[chunk, pl.ds(sid * nz, nz)])
            plsc.subcore_barrier()

    return k(table, s3, d3, zrows)


def _deg_call(d3, ones_rows, zrows, n_dst):
    n_dst_pad = _pad128(n_dst)
    nz = n_dst_pad // 16
    nbt = _NB // 32
    mesh = plsc.VectorSubcoreMesh(core_axis_name="c", subcore_axis_name="s")

    @functools.partial(
        pl.kernel,
        out_type=jax.ShapeDtypeStruct((2, n_dst_pad, 16), jnp.float32),
        mesh=mesh,
        scratch_types=[
            pltpu.VMEM((_K, 128), jnp.int32),
            pltpu.VMEM((128, 16), jnp.float32),
            pltpu.VMEM_SHARED((n_dst_pad, 16), jnp.float32),
        ],
        compiler_params=pltpu.CompilerParams(use_tc_tiling_on_sc=False),
    )
    def k(d_idx, ones_hbm, zr, out, dbuf, obuf, acc):
        cid = lax.axis_index("c")
        sid = lax.axis_index("s")
        pltpu.sync_copy(ones_hbm, obuf)
        pltpu.sync_copy(zr, acc.at[pl.ds(sid * nz, nz)])
        plsc.subcore_barrier()

        def body(i, carry):
            b = (cid * 16 + sid) * nbt + i
            pltpu.sync_copy(d_idx.at[b], dbuf)
            for kk in range(_K):
                pltpu.sync_copy(obuf, acc.at[dbuf.at[kk]], add=True)
            return carry

        lax.fori_loop(0, nbt, body, 0)
        plsc.subcore_barrier()
        pltpu.sync_copy(acc.at[pl.ds(sid * nz, nz)],
                        out.at[cid, pl.ds(sid * nz, nz)])
        plsc.subcore_barrier()

    return k(d3, ones_rows, zrows)


def _proj_body(x_ref, w_ref, o_ref):
    o_ref[...] = jnp.dot(x_ref[...], w_ref[0],
                         preferred_element_type=jnp.float32)[None]


def _proj_call(x, w):
    n = x.shape[0]
    nb = n // _R
    wc = w.reshape(_D, _NCHUNK, _CW).transpose(1, 0, 2)
    return pl.pallas_call(
        _proj_body,
        grid=(nb, _NCHUNK),
        in_specs=[
            pl.BlockSpec((_R, _D), lambda nn, cc: (nn, 0)),
            pl.BlockSpec((1, _D, _CW), lambda nn, cc: (cc, 0, 0)),
        ],
        out_specs=pl.BlockSpec((1, _R, _CW), lambda nn, cc: (cc, nn, 0)),
        out_shape=jax.ShapeDtypeStruct((_NCHUNK, n, _CW), jnp.float32),
    )(x, wc)


def _inv_deg(dg):
    deg = dg[0, :, 0:1] + dg[1, :, 0:1]
    return 1.0 / jnp.maximum(deg, 1.0)


def _neigh(s0, s1, s2, s3, dg):
    full = jnp.concatenate([s0[0], s1[0], s2[0], s3[0]], axis=1)
    return full * _inv_deg(dg[...])


def _combine1_body(x_ref, ws_ref, b_ref, s0, s1, s2, s3, dg, o_ref):
    acc = jnp.dot(x_ref[...], ws_ref[...], preferred_element_type=jnp.float32)
    acc = acc + b_ref[...] + _neigh(s0[...], s1[...], s2[...], s3[...], dg)
    o_ref[...] = jnp.maximum(acc, 0.0)


def _combine2_body(x_ref, wsa_ref, wsb_ref, ba_ref, bb_ref,
                   a0, a1, a2, a3, dga, b0, b1, b2, b3, dgb, o_ref):
    ws = wsa_ref[...] + wsb_ref[...]
    acc = jnp.dot(x_ref[...], ws, preferred_element_type=jnp.float32)
    acc = acc + ba_ref[...] + bb_ref[...]
    acc = acc + _neigh(a0[...], a1[...], a2[...], a3[...], dga)
    acc = acc + _neigh(b0[...], b1[...], b2[...], b3[...], dgb)
    o_ref[...] = jnp.maximum(acc, 0.0)


def _combine2_head_body(x_ref, wsa_ref, wsb_ref, ba_ref, bb_ref,
                        a0, a1, a2, a3, dga, b0, b1, b2, b3, dgb,
                        w1_ref, b1_ref, w2_ref, b2_ref, o_ref):
    ws = wsa_ref[...] + wsb_ref[...]
    acc = jnp.dot(x_ref[...], ws, preferred_element_type=jnp.float32)
    acc = acc + ba_ref[...] + bb_ref[...]
    acc = acc + _neigh(a0[...], a1[...], a2[...], a3[...], dga)
    acc = acc + _neigh(b0[...], b1[...], b2[...], b3[...], dgb)
    h = jnp.maximum(acc, 0.0)
    z = jnp.maximum(jnp.dot(h, w1_ref[...],
                            preferred_element_type=jnp.float32) + b1_ref[...],
                    0.0)
    o_ref[...] = jnp.dot(z, w2_ref[...],
                         preferred_element_type=jnp.float32) + b2_ref[...]


def _seg_specs(n_dst_pad):
    specs = []
    for c in range(_NCHUNK):
        specs.append(pl.BlockSpec((1, _R, _CW), lambda nn, c=c: (c, nn, 0)))
    specs.append(pl.BlockSpec((2, _R, 16), lambda nn: (0, nn, 0)))
    return specs


def _combine1_call(x, ws, b, seg, dg):
    n = x.shape[0]
    nb = n // _R
    in_specs = [
        pl.BlockSpec((_R, _D), lambda nn: (nn, 0)),
        pl.BlockSpec((_D, _D), lambda nn: (0, 0)),
        pl.BlockSpec((1, _D), lambda nn: (0, 0)),
    ] + _seg_specs(seg.shape[1])
    return pl.pallas_call(
        _combine1_body,
        grid=(nb,),
        in_specs=in_specs,
        out_specs=pl.BlockSpec((_R, _D), lambda nn: (nn, 0)),
        out_shape=jax.ShapeDtypeStruct((n, _D), jnp.float32),
    )(x, ws, b, seg, seg, seg, seg, dg)


def _combine2_call(x, wsa, wsb, ba, bb, sega, dga, segb, dgb,
                   head=None):
    n = x.shape[0]
    nb = n // _R
    in_specs = [
        pl.BlockSpec((_R, _D), lambda nn: (nn, 0)),
        pl.BlockSpec((_D, _D), lambda nn: (0, 0)),
        pl.BlockSpec((_D, _D), lambda nn: (0, 0)),
        pl.BlockSpec((1, _D), lambda nn: (0, 0)),
        pl.BlockSpec((1, _D), lambda nn: (0, 0)),
    ] + _seg_specs(sega.shape[1]) + _seg_specs(segb.shape[1])
    args = [x, wsa, wsb, ba, bb, sega, sega, sega, sega, dga,
            segb, segb, segb, segb, dgb]
    if head is None:
        return pl.pallas_call(
            _combine2_body,
            grid=(nb,),
            in_specs=in_specs,
            out_specs=pl.BlockSpec((_R, _D), lambda nn: (nn, 0)),
            out_shape=jax.ShapeDtypeStruct((n, _D), jnp.float32),
        )(*args)
    w1, b1, w2, b2 = head
    in_specs = in_specs + [
        pl.BlockSpec((_D, _D), lambda nn: (0, 0)),
        pl.BlockSpec((1, _D), lambda nn: (0, 0)),
        pl.BlockSpec((_D, 8), lambda nn: (0, 0)),
        pl.BlockSpec((1, 8), lambda nn: (0, 0)),
    ]
    return pl.pallas_call(
        _combine2_head_body,
        grid=(nb,),
        in_specs=in_specs,
        out_specs=pl.BlockSpec((_R, 8), lambda nn: (nn, 0)),
        out_shape=jax.ShapeDtypeStruct((n, 8), jnp.float32),
    )(*(args + [w1, b1, w2, b2]))


def _pad_edges(s, d, trash):
    e = s.shape[0]
    pad = _EP - e
    s3 = jnp.concatenate(
        [s.astype(jnp.int32), jnp.zeros((pad,), jnp.int32)]).reshape(_NB, _K, 128)
    d3 = jnp.concatenate(
        [d.astype(jnp.int32),
         jnp.full((pad,), trash, jnp.int32)]).reshape(_NB, _K, 128)
    return s3, d3


def kernel(tx_feats, emb_card, emb_merch, tc_src, tc_dst, tm_src, tm_dst,
           params):
    n_tx, n_card, n_merch = tx_feats.shape[0], emb_card.shape[0], emb_merch.shape[0]
    p = params

    tc_s3, tc_d3 = _pad_edges(tc_src, tc_dst, n_card)
    ct_s3, ct_d3 = _pad_edges(tc_dst, tc_src, n_tx)
    tm_s3, tm_d3 = _pad_edges(tm_src, tm_dst, n_merch)
    mt_s3, mt_d3 = _pad_edges(tm_dst, tm_src, n_tx)

    z32 = {n: jnp.zeros((_pad128(n) // 16, _CW), jnp.float32)
           for n in (n_tx, n_card, n_merch)}
    z16 = {n: jnp.zeros((_pad128(n) // 16, 16), jnp.float32)
           for n in (n_tx, n_card, n_merch)}
    ones16 = jnp.ones((128, 16), jnp.float32)

    deg_card = _deg_call(tc_d3, ones16, z16[n_card], n_card)
    deg_tx_c = _deg_call(ct_d3, ones16, z16[n_tx], n_tx)
    deg_merch = _deg_call(tm_d3, ones16, z16[n_merch], n_merch)
    deg_tx_m = _deg_call(mt_d3, ones16, z16[n_tx], n_tx)

    h_tx, h_card, h_merch = tx_feats, emb_card, emb_merch
    bias = {k: v.reshape(1, _D) for k, v in p.items() if k.startswith('b_')}

    for l in range(2):
        p_tx_tc = _proj_call(h_tx, p['W_neigh_%d_tc' % l])
        p_tx_tm = _proj_call(h_tx, p['W_neigh_%d_tm' % l])
        p_card = _proj_call(h_card, p['W_neigh_%d_ct' % l])
        p_merch = _proj_call(h_merch, p['W_neigh_%d_mt' % l])

        seg_card = _segsum_call(p_tx_tc, tc_s3, tc_d3, z32[n_card], n_tx, n_card)
        seg_merch = _segsum_call(p_tx_tm, tm_s3, tm_d3, z32[n_merch], n_tx, n_merch)
        seg_tx_c = _segsum_call(p_card, ct_s3, ct_d3, z32[n_tx], n_card, n_tx)
        seg_tx_m = _segsum_call(p_merch, mt_s3, mt_d3, z32[n_tx], n_merch, n_tx)

        new_card = _combine1_call(h_card, p['W_self_%d_tc' % l],
                                  bias['b_%d_tc' % l], seg_card, deg_card)
        new_merch = _combine1_call(h_merch, p['W_self_%d_tm' % l],
                                   bias['b_%d_tm' % l], seg_merch, deg_merch)
        head = None
        if l == 1:
            w2p = jnp.pad(p['head_W2'], ((0, 0), (0, 7)))
            b2p = jnp.pad(p['head_b2'].reshape(1, 1), ((0, 0), (0, 7)))
            head = (p['head_W1'], p['head_b1'].reshape(1, _D), w2p, b2p)
        new_tx = _combine2_call(h_tx, p['W_self_%d_ct' % l],
                                p['W_self_%d_mt' % l],
                                bias['b_%d_ct' % l], bias['b_%d_mt' % l],
                                seg_tx_c, deg_tx_c, seg_tx_m, deg_tx_m,
                                head=head)
        h_tx, h_card, h_merch = new_tx, new_card, new_merch

    return h_tx[:, 0]
